# baseline (speedup 1.0000x reference)
import functools
import jax
import jax.numpy as jnp
from jax import lax
from jax.experimental import pallas as pl
from jax.experimental.pallas import tpu as pltpu
from jax.experimental.pallas import tpu_sc as plsc

_N = 10000
_D = 128
_NC, _NS = 2, 16
_NW = _NC * _NS
_CH = 128
_ZROWS = 632
_ACC_ROWS = _NS * _ZROWS


def _scatter_partials(h, sd, zeros, cpw):
    mesh = plsc.VectorSubcoreMesh(core_axis_name="c", subcore_axis_name="s")

    @functools.partial(
        pl.kernel,
        out_type=jax.ShapeDtypeStruct((_NC * _ACC_ROWS, _D), jnp.float32),
        mesh=mesh,
        scratch_types=[
            pltpu.VMEM((2, _CH), jnp.int32),
            pltpu.VMEM((2, _CH), jnp.int32),
            pltpu.VMEM((_CH, _D), jnp.float32),
            pltpu.VMEM_SHARED((_ACC_ROWS, _D), jnp.float32),
            pltpu.SemaphoreType.DMA,
            pltpu.SemaphoreType.DMA,
            pltpu.SemaphoreType.DMA,
        ],
    )
    def k(h_hbm, sd_hbm, zeros_hbm, out_hbm, sd0, sd1, rows_v, acc,
          sem, st0, st1):
        sds = (sd0, sd1)
        sts = (st0, st1)
        cid = lax.axis_index("c")
        sid = lax.axis_index("s")
        wid = cid * _NS + sid
        pltpu.sync_copy(zeros_hbm, acc.at[pl.ds(sid * _ZROWS, _ZROWS)])
        plsc.subcore_barrier()
        base = wid * cpw

        pltpu.async_copy(sd_hbm.at[base], sds[0], sts[0])

        def body(g, carry):
            i = g * 2
            for b in range(2):
                j = i + b
                o = 1 - b
                pltpu.make_async_copy(sd_hbm.at[base], sds[b], sts[b]).wait()

                @pl.when(j + 1 < cpw)
                def _():
                    pltpu.async_copy(sd_hbm.at[base + j + 1], sds[o], sts[o])

                pltpu.async_copy(h_hbm.at[sds[b].at[0]], rows_v, sem).wait()
                pltpu.sync_copy(rows_v, acc.at[sds[b].at[1]], add=True)
            return carry

        lax.fori_loop(0, cpw // 2, body, 0)
        plsc.subcore_barrier()
        pltpu.sync_copy(
            acc.at[pl.ds(sid * _ZROWS, _ZROWS)],
            out_hbm.at[pl.ds(cid * _ACC_ROWS + sid * _ZROWS, _ZROWS)],
        )

    return k(h, sd, zeros)


def _mlp1_body(x_ref, p0_ref, p1_ref, w1a_ref, b1a_ref, w1b_ref, b1b_ref,
               w2a_ref, u_ref):
    z = x_ref[...] + p0_ref[...] + p1_ref[...]
    y = jnp.maximum(
        jnp.dot(z, w1a_ref[...], preferred_element_type=jnp.float32)
        + b1a_ref[...], 0.0)
    h1 = jnp.maximum(
        jnp.dot(y, w1b_ref[...], preferred_element_type=jnp.float32)
        + b1b_ref[...], 0.0)
    u_ref[...] = jnp.dot(h1, w2a_ref[...], preferred_element_type=jnp.float32)


def _mlp2_body(u_ref, q0_ref, q1_ref, b2a_ref, w2b_ref, b2b_ref, o_ref):
    s = jnp.maximum(u_ref[...] + q0_ref[...] + q1_ref[...] + b2a_ref[...], 0.0)
    o_ref[...] = (
        jnp.dot(s, w2b_ref[...], preferred_element_type=jnp.float32)
        + b2b_ref[...])


_BN = 2000


def _row_spec(d):
    return pl.BlockSpec((_BN, d), lambda i: (i, 0))


def _full_spec(r, c):
    return pl.BlockSpec((r, c), lambda i: (0, 0))


def kernel(x, edge_index, W1a, b1a, W1b, b1b, W2a, b2a, W2b, b2b):
    src = edge_index[0]
    dst = edge_index[1]
    E = src.shape[0]
    chunks = -(-E // _CH)
    cpw = -(-chunks // _NW)
    cpw += cpw % 2
    pad = cpw * _NW * _CH - E
    srcp = jnp.concatenate([src, jnp.zeros((pad,), jnp.int32)])
    dummy_dst = _N + jnp.arange(pad, dtype=jnp.int32) % (_ACC_ROWS - _N)
    dstp = jnp.concatenate([dst, dummy_dst])
    sd = jnp.stack([srcp.reshape(-1, _CH), dstp.reshape(-1, _CH)], axis=1)
    zeros = jnp.zeros((_ZROWS, _D), jnp.float32)

    parts1 = _scatter_partials(x, sd, zeros, cpw)
    p0, p1 = parts1[:_N], parts1[_ACC_ROWS:_ACC_ROWS + _N]

    grid = _N // _BN
    u = pl.pallas_call(
        _mlp1_body,
        grid=(grid,),
        in_specs=[
            _row_spec(_D), _row_spec(_D), _row_spec(_D),
            _full_spec(_D, 2 * _D), _full_spec(1, 2 * _D),
            _full_spec(2 * _D, 2 * _D), _full_spec(1, 2 * _D),
            _full_spec(2 * _D, _D),
        ],
        out_specs=_row_spec(_D),
        out_shape=jax.ShapeDtypeStruct((_N, _D), jnp.float32),
    )(x, p0, p1, W1a, b1a.reshape(1, -1), W1b, b1b.reshape(1, -1), W2a)

    parts2 = _scatter_partials(u, sd, zeros, cpw)
    q0, q1 = parts2[:_N], parts2[_ACC_ROWS:_ACC_ROWS + _N]

    out = pl.pallas_call(
        _mlp2_body,
        grid=(grid,),
        in_specs=[
            _row_spec(_D), _row_spec(_D), _row_spec(_D),
            _full_spec(1, _D), _full_spec(_D, _D), _full_spec(1, _D),
        ],
        out_specs=_row_spec(_D),
        out_shape=jax.ShapeDtypeStruct((_N, _D), jnp.float32),
    )(u, q0, q1, b2a.reshape(1, -1), W2b, b2b.reshape(1, -1))
    return out


# speedup vs baseline: 1.6159x; 1.6159x over previous
import functools
import jax
import jax.numpy as jnp
from jax import lax
from jax.experimental import pallas as pl
from jax.experimental.pallas import tpu as pltpu
from jax.experimental.pallas import tpu_sc as plsc

_N = 10000
_D = 128
_NC, _NS = 2, 16
_NW = _NC * _NS
_CH = 128
_ZROWS = 632
_ACC_ROWS = _NS * _ZROWS


def _scatter_partials(h, sd, zeros, cpw):
    mesh = plsc.VectorSubcoreMesh(core_axis_name="c", subcore_axis_name="s")

    @functools.partial(
        pl.kernel,
        out_type=jax.ShapeDtypeStruct((_NC * _ACC_ROWS, _D), jnp.float32),
        mesh=mesh,
        scratch_types=[
            pltpu.VMEM((cpw, 2, _CH), jnp.int32),
            pltpu.VMEM((_CH, _D), jnp.float32),
            pltpu.VMEM_SHARED((_ACC_ROWS, _D), jnp.float32),
            pltpu.SemaphoreType.DMA,
        ],
    )
    def k(h_hbm, sd_hbm, zeros_hbm, out_hbm, sd_v, rows_v, acc, sem):
        cid = lax.axis_index("c")
        sid = lax.axis_index("s")
        wid = cid * _NS + sid
        pltpu.sync_copy(zeros_hbm, acc.at[pl.ds(sid * _ZROWS, _ZROWS)])
        pltpu.sync_copy(sd_hbm.at[pl.ds(wid * cpw, cpw)], sd_v)
        plsc.subcore_barrier()

        def body(j, carry):
            pltpu.async_copy(h_hbm.at[sd_v.at[j].at[0]], rows_v, sem).wait()
            pltpu.sync_copy(rows_v, acc.at[sd_v.at[j].at[1]], add=True)
            return carry

        lax.fori_loop(0, cpw, body, 0)
        plsc.subcore_barrier()
        pltpu.sync_copy(
            acc.at[pl.ds(sid * _ZROWS, _ZROWS)],
            out_hbm.at[pl.ds(cid * _ACC_ROWS + sid * _ZROWS, _ZROWS)],
        )

    return k(h, sd, zeros)


def _mlp1_body(x_ref, p0_ref, p1_ref, w1a_ref, b1a_ref, w1b_ref, b1b_ref,
               w2a_ref, u_ref):
    z = x_ref[...] + p0_ref[...] + p1_ref[...]
    y = jnp.maximum(
        jnp.dot(z, w1a_ref[...], preferred_element_type=jnp.float32)
        + b1a_ref[...], 0.0)
    h1 = jnp.maximum(
        jnp.dot(y, w1b_ref[...], preferred_element_type=jnp.float32)
        + b1b_ref[...], 0.0)
    u_ref[...] = jnp.dot(h1, w2a_ref[...], preferred_element_type=jnp.float32)


def _mlp2_body(u_ref, q0_ref, q1_ref, b2a_ref, w2b_ref, b2b_ref, o_ref):
    s = jnp.maximum(u_ref[...] + q0_ref[...] + q1_ref[...] + b2a_ref[...], 0.0)
    o_ref[...] = (
        jnp.dot(s, w2b_ref[...], preferred_element_type=jnp.float32)
        + b2b_ref[...])


_BN = 2000


def _row_spec(d):
    return pl.BlockSpec((_BN, d), lambda i: (i, 0))


def _full_spec(r, c):
    return pl.BlockSpec((r, c), lambda i: (0, 0))


def kernel(x, edge_index, W1a, b1a, W1b, b1b, W2a, b2a, W2b, b2b):
    src = edge_index[0]
    dst = edge_index[1]
    E = src.shape[0]
    chunks = -(-E // _CH)
    cpw = -(-chunks // _NW)
    pad = cpw * _NW * _CH - E
    srcp = jnp.concatenate([src, jnp.zeros((pad,), jnp.int32)])
    dummy_dst = _N + jnp.arange(pad, dtype=jnp.int32) % (_ACC_ROWS - _N)
    dstp = jnp.concatenate([dst, dummy_dst])
    sd = jnp.stack([srcp.reshape(-1, _CH), dstp.reshape(-1, _CH)], axis=1)
    zeros = jnp.zeros((_ZROWS, _D), jnp.float32)

    parts1 = _scatter_partials(x, sd, zeros, cpw)
    p0, p1 = parts1[:_N], parts1[_ACC_ROWS:_ACC_ROWS + _N]

    grid = _N // _BN
    u = pl.pallas_call(
        _mlp1_body,
        grid=(grid,),
        in_specs=[
            _row_spec(_D), _row_spec(_D), _row_spec(_D),
            _full_spec(_D, 2 * _D), _full_spec(1, 2 * _D),
            _full_spec(2 * _D, 2 * _D), _full_spec(1, 2 * _D),
            _full_spec(2 * _D, _D),
        ],
        out_specs=_row_spec(_D),
        out_shape=jax.ShapeDtypeStruct((_N, _D), jnp.float32),
    )(x, p0, p1, W1a, b1a.reshape(1, -1), W1b, b1b.reshape(1, -1), W2a)

    parts2 = _scatter_partials(u, sd, zeros, cpw)
    q0, q1 = parts2[:_N], parts2[_ACC_ROWS:_ACC_ROWS + _N]

    out = pl.pallas_call(
        _mlp2_body,
        grid=(grid,),
        in_specs=[
            _row_spec(_D), _row_spec(_D), _row_spec(_D),
            _full_spec(1, _D), _full_spec(_D, _D), _full_spec(1, _D),
        ],
        out_specs=_row_spec(_D),
        out_shape=jax.ShapeDtypeStruct((_N, _D), jnp.float32),
    )(u, q0, q1, b2a.reshape(1, -1), W2b, b2b.reshape(1, -1))
    return out
